# baseline (speedup 1.0000x reference)
import functools

import jax
import jax.numpy as jnp
import numpy as np
from jax import lax
from jax.experimental import pallas as pl
from jax.experimental.pallas import tpu as pltpu
from jax.experimental.pallas import tpu_sc as plsc

N_ENTITY = 1000000
DIM = 64
BATCH = 16384

_NTC = 4096
_NSC = BATCH - _NTC

_info = plsc.get_sparse_core_info()
_NC = _info.num_cores
_NS = _info.num_subcores
_NW = _NC * _NS
_BPW = _NSC // _NW
_L = _info.num_lanes
_RING = 8
_GRP = 16


def _extract(blk_slot, l, b, col_v):
    lvec = jnp.full((_L,), l, dtype=jnp.int32)
    bvec = jnp.full((_L,), b, dtype=jnp.int32)
    for k in range(DIM // _L):
        dvec = lax.iota(jnp.int32, _L) + k * _L
        vals = plsc.load_gather(blk_slot, [dvec, lvec])
        plsc.store_scatter(col_v, [dvec, bvec], vals)


def _gather_body(idx_hbm, muT_hbm, outT_hbm, idx_v, blk_v, col_v, lsm, sems):
    wid = lax.axis_index("s") * _NC + lax.axis_index("c")
    base = wid * _BPW
    pltpu.sync_copy(idx_hbm.at[pl.ds(base, _BPW)], idx_v)

    def group(g, _):
        v16 = idx_v[pl.ds(g * _GRP, _GRP)]
        for j in range(_GRP):
            m = g * _GRP + j
            slot = j % _RING
            i = jnp.squeeze(lax.slice(v16, (j,), (j + 1,)))
            c = pl.multiple_of((i >> 7) << 7, 128)
            l = i & 127

            @pl.when(m >= _RING)
            def _():
                pltpu.make_async_copy(
                    muT_hbm.at[:, pl.ds(0, 128)], blk_v.at[slot], sems[slot]
                ).wait()
                _extract(blk_v.at[slot], lsm[slot], m - _RING, col_v)

            pltpu.async_copy(
                muT_hbm.at[:, pl.ds(c, 128)], blk_v.at[slot], sems[slot]
            )
            lsm[slot] = l
        return _

    lax.fori_loop(0, _BPW // _GRP, group, None)
    for jj in range(_RING):
        pltpu.make_async_copy(
            muT_hbm.at[:, pl.ds(0, 128)], blk_v.at[jj], sems[jj]
        ).wait()
        _extract(blk_v.at[jj], lsm[jj], _BPW - _RING + jj, col_v)

    pltpu.sync_copy(col_v, outT_hbm.at[:, pl.ds(base, _BPW)])


@functools.partial(
    pl.kernel,
    out_type=jax.ShapeDtypeStruct((DIM, _NSC), jnp.float32),
    mesh=plsc.VectorSubcoreMesh(core_axis_name="c", subcore_axis_name="s"),
    scratch_types=[
        pltpu.VMEM((_BPW,), jnp.int32),
        pltpu.VMEM((_RING, DIM, 128), jnp.float32),
        pltpu.VMEM((DIM, _BPW), jnp.float32),
        pltpu.SMEM((_RING,), jnp.int32),
    ] + [pltpu.SemaphoreType.DMA] * _RING,
    compiler_params=pltpu.CompilerParams(
        use_tc_tiling_on_sc=True,
        needs_layout_passes=False,
        disable_bounds_checks=True,
    ),
)
def _sc_gather(idx_hbm, muT_hbm, outT_hbm, idx_v, blk_v, col_v, lsm, *sems):
    _gather_body(idx_hbm, muT_hbm, outT_hbm, idx_v, blk_v, col_v, lsm,
                 list(sems))


_TCHUNK = 128
_TRING = 16


def _tc_body(idx_sm, muT_any, out_ref, blk_v, sems):
    ch = pl.program_id(0)

    def start(j):
        i = idx_sm[ch * _TCHUNK + j]
        c = pl.multiple_of((i >> 7) << 7, 128)
        pltpu.make_async_copy(
            muT_any.at[:, pl.ds(c, 128)], blk_v.at[j % _TRING],
            sems.at[j % _TRING],
        ).start()

    for j in range(_TRING):
        start(j)

    lanes = lax.broadcasted_iota(jnp.int32, (1, 128), 1)
    for j in range(_TCHUNK):
        pltpu.make_async_copy(
            muT_any.at[:, pl.ds(0, 128)], blk_v.at[j % _TRING],
            sems.at[j % _TRING],
        ).wait()
        i = idx_sm[ch * _TCHUNK + j]
        l = i & 127
        blk = blk_v[j % _TRING]
        onehot = (lanes == l).astype(jnp.float32)
        col = lax.dot_general(
            blk, onehot, (((1,), (1,)), ((), ())),
            preferred_element_type=jnp.float32,
        )
        out_ref[:, pl.ds(j, 1)] = col
        if j + _TRING < _TCHUNK:
            start(j + _TRING)


_tc_gather = pl.pallas_call(
    _tc_body,
    grid=(_NTC // _TCHUNK,),
    in_specs=[
        pl.BlockSpec(memory_space=pltpu.SMEM),
        pl.BlockSpec(memory_space=pl.ANY),
    ],
    out_specs=pl.BlockSpec((DIM, _TCHUNK), lambda i: (0, i)),
    out_shape=jax.ShapeDtypeStruct((DIM, _NTC), jnp.float32),
    scratch_shapes=[
        pltpu.VMEM((_TRING, DIM, 128), jnp.float32),
        pltpu.SemaphoreType.DMA((_TRING,)),
    ],
)


def kernel(idx, mu):
    idx = idx.astype(jnp.int32)
    muT = mu.T
    out_sc = _sc_gather(idx[:_NSC], muT)
    out_tc = _tc_gather(idx[_NSC:], muT)
    return jnp.concatenate([out_sc, out_tc], axis=1).T


# speedup vs baseline: 1.7888x; 1.7888x over previous
import functools

import jax
import jax.numpy as jnp
import numpy as np
from jax import lax
from jax.experimental import pallas as pl
from jax.experimental.pallas import tpu as pltpu
from jax.experimental.pallas import tpu_sc as plsc

N_ENTITY = 1000000
DIM = 64
BATCH = 16384

_info = plsc.get_sparse_core_info()
_NC = _info.num_cores
_NS = _info.num_subcores
_NW = _NC * _NS
_BPW = BATCH // _NW
_L = _info.num_lanes
_RING = 8
_GRP = 16
_HD = DIM // 2


def _extract_half(blk_slot, l, b, col_v, half):
    lvec = jnp.full((_L,), l, dtype=jnp.int32)
    bvec = jnp.full((_L,), b, dtype=jnp.int32)
    for k in range(_HD // _L):
        dvec = lax.iota(jnp.int32, _L) + k * _L
        vals = plsc.load_gather(blk_slot, [dvec, lvec])
        plsc.store_scatter(col_v, [dvec + half * _HD, bvec], vals)


def _gather_body(idx_hbm, muT_hbm, outT_hbm, idx_v, blkA, blkB, col_v, lsm,
                 semsA, semsB):
    wid = lax.axis_index("s") * _NC + lax.axis_index("c")
    base = wid * _BPW
    pltpu.sync_copy(idx_hbm.at[pl.ds(base, _BPW)], idx_v)

    def group(g, _):
        v16 = idx_v[pl.ds(g * _GRP, _GRP)]
        for j in range(_GRP):
            m = g * _GRP + j
            slot = j % _RING
            i = jnp.squeeze(lax.slice(v16, (j,), (j + 1,)))
            c = pl.multiple_of((i >> 7) << 7, 128)
            l = i & 127

            @pl.when(m >= _RING)
            def _():
                pltpu.make_async_copy(
                    muT_hbm.at[pl.ds(0, _HD), pl.ds(0, 128)],
                    blkA.at[slot], semsA[slot],
                ).wait()
                pltpu.make_async_copy(
                    muT_hbm.at[pl.ds(0, _HD), pl.ds(0, 128)],
                    blkB.at[slot], semsB[slot],
                ).wait()
                _extract_half(blkA.at[slot], lsm[slot], m - _RING, col_v, 0)
                _extract_half(blkB.at[slot], lsm[slot], m - _RING, col_v, 1)

            pltpu.async_copy(
                muT_hbm.at[pl.ds(0, _HD), pl.ds(c, 128)], blkA.at[slot],
                semsA[slot],
            )
            pltpu.async_copy(
                muT_hbm.at[pl.ds(_HD, _HD), pl.ds(c, 128)], blkB.at[slot],
                semsB[slot],
            )
            lsm[slot] = l
        return _

    lax.fori_loop(0, _BPW // _GRP, group, None)
    for jj in range(_RING):
        pltpu.make_async_copy(
            muT_hbm.at[pl.ds(0, _HD), pl.ds(0, 128)], blkA.at[jj], semsA[jj]
        ).wait()
        pltpu.make_async_copy(
            muT_hbm.at[pl.ds(0, _HD), pl.ds(0, 128)], blkB.at[jj], semsB[jj]
        ).wait()
        _extract_half(blkA.at[jj], lsm[jj], _BPW - _RING + jj, col_v, 0)
        _extract_half(blkB.at[jj], lsm[jj], _BPW - _RING + jj, col_v, 1)

    pltpu.sync_copy(col_v, outT_hbm.at[:, pl.ds(base, _BPW)])


@functools.partial(
    pl.kernel,
    out_type=jax.ShapeDtypeStruct((DIM, BATCH), jnp.float32),
    mesh=plsc.VectorSubcoreMesh(core_axis_name="c", subcore_axis_name="s"),
    scratch_types=[
        pltpu.VMEM((_BPW,), jnp.int32),
        pltpu.VMEM((_RING, _HD, 128), jnp.float32),
        pltpu.VMEM((_RING, _HD, 128), jnp.float32),
        pltpu.VMEM((DIM, _BPW), jnp.float32),
        pltpu.SMEM((_RING,), jnp.int32),
    ] + [pltpu.SemaphoreType.DMA] * (2 * _RING),
    compiler_params=pltpu.CompilerParams(
        use_tc_tiling_on_sc=True,
        needs_layout_passes=False,
        disable_bounds_checks=True,
    ),
)
def _sc_gather(idx_hbm, muT_hbm, outT_hbm, idx_v, blkA, blkB, col_v, lsm,
               *sems):
    _gather_body(idx_hbm, muT_hbm, outT_hbm, idx_v, blkA, blkB, col_v, lsm,
                 list(sems[:_RING]), list(sems[_RING:]))


def kernel(idx, mu):
    return _sc_gather(idx.astype(jnp.int32), mu.T).T


# speedup vs baseline: 1.9736x; 1.1033x over previous
import functools

import jax
import jax.numpy as jnp
import numpy as np
from jax import lax
from jax.experimental import pallas as pl
from jax.experimental.pallas import tpu as pltpu
from jax.experimental.pallas import tpu_sc as plsc

N_ENTITY = 1000000
DIM = 64
BATCH = 16384

_info = plsc.get_sparse_core_info()
_NC = _info.num_cores
_NS = _info.num_subcores
_NW = _NC * _NS
_BPW = BATCH // _NW
_L = _info.num_lanes
_RING = 8
_GRP = 16


def _extract(blk_slot, l, b, col_v):
    lvec = jnp.full((_L,), l, dtype=jnp.int32)
    bvec = jnp.full((_L,), b, dtype=jnp.int32)
    for k in range(DIM // _L):
        dvec = lax.iota(jnp.int32, _L) + k * _L
        vals = plsc.load_gather(blk_slot, [dvec, lvec])
        plsc.store_scatter(col_v, [dvec, bvec], vals)


def _gather_body(idx_hbm, muT_hbm, outT_hbm, idx_v, blk_v, col_v, lsm, sems):
    wid = lax.axis_index("s") * _NC + lax.axis_index("c")
    base = wid * _BPW
    pltpu.sync_copy(idx_hbm.at[pl.ds(base, _BPW)], idx_v)

    def group(g, _):
        v16 = idx_v[pl.ds(g * _GRP, _GRP)]
        for j in range(_GRP):
            m = g * _GRP + j
            slot = j % _RING
            i = jnp.squeeze(lax.slice(v16, (j,), (j + 1,)))
            c = pl.multiple_of((i >> 7) << 7, 128)
            l = i & 127

            @pl.when(m >= _RING)
            def _():
                pltpu.make_async_copy(
                    muT_hbm.at[:, pl.ds(0, 128)], blk_v.at[slot], sems[slot]
                ).wait()
                _extract(blk_v.at[slot], lsm[slot], m - _RING, col_v)

            pltpu.async_copy(
                muT_hbm.at[:, pl.ds(c, 128)], blk_v.at[slot], sems[slot]
            )
            lsm[slot] = l
        return _

    lax.fori_loop(0, _BPW // _GRP, group, None)
    for jj in range(_RING):
        pltpu.make_async_copy(
            muT_hbm.at[:, pl.ds(0, 128)], blk_v.at[jj], sems[jj]
        ).wait()
        _extract(blk_v.at[jj], lsm[jj], _BPW - _RING + jj, col_v)

    pltpu.sync_copy(col_v, outT_hbm.at[:, pl.ds(base, _BPW)])


@functools.partial(
    pl.kernel,
    out_type=jax.ShapeDtypeStruct((DIM, BATCH), jnp.float32),
    mesh=plsc.VectorSubcoreMesh(core_axis_name="c", subcore_axis_name="s"),
    scratch_types=[
        pltpu.VMEM((_BPW,), jnp.int32),
        pltpu.VMEM((_RING, DIM, 128), jnp.float32),
        pltpu.VMEM((DIM, _BPW), jnp.float32),
        pltpu.SMEM((_RING,), jnp.int32),
    ] + [pltpu.SemaphoreType.DMA] * _RING,
    compiler_params=pltpu.CompilerParams(
        use_tc_tiling_on_sc=True,
        needs_layout_passes=False,
        disable_bounds_checks=True,
    ),
)
def _sc_gather(idx_hbm, muT_hbm, outT_hbm, idx_v, blk_v, col_v, lsm, *sems):
    _gather_body(idx_hbm, muT_hbm, outT_hbm, idx_v, blk_v, col_v, lsm,
                 list(sems))


def kernel(idx, mu):
    return _sc_gather(idx.astype(jnp.int32), mu.T).T
